# baseline (speedup 1.0000x reference)
import functools

import jax
import jax.numpy as jnp
from jax import lax
from jax.experimental import pallas as pl
from jax.experimental.pallas import tpu as pltpu
from jax.experimental.pallas import tpu_sc as plsc

_B, _S, _D = 16, 4096, 512
_NT = 32
_NO = 32
_NTH = 16
_DD = 128
_ROW = 2 * _D + _DD
_BINS = (1, 2, 3, 4, 5, 8, 16, 32, 64)

_mesh = plsc.VectorSubcoreMesh(core_axis_name="c", subcore_axis_name="s")


@functools.partial(
    pl.kernel,
    mesh=_mesh,
    compiler_params=pltpu.CompilerParams(needs_layout_passes=False),
    out_type=jax.ShapeDtypeStruct((_B * _NT * _NO, _ROW), jnp.float32),
    scratch_types=[
        pltpu.VMEM((2 * _S,), jnp.int32),
        pltpu.VMEM((_NTH,), jnp.int32),
        pltpu.VMEM((_NO,), jnp.int32),
        pltpu.VMEM((_NTH,), jnp.int32),
        pltpu.VMEM((_NO,), jnp.int32),
        pltpu.VMEM((2 * _NTH,), jnp.int32),
        pltpu.VMEM((2, _NO), jnp.int32),
        pltpu.VMEM((_NTH, _D), jnp.float32),
        pltpu.VMEM((_NO, _D), jnp.float32),
        pltpu.VMEM((2, _NO, _D), jnp.float32),
        pltpu.VMEM((2, _NO, _DD), jnp.float32),
        pltpu.SemaphoreType.DMA,
        pltpu.SemaphoreType.DMA,
        pltpu.SemaphoreType.DMA,
    ],
)
def _pair_rep_sc(spans2d, sidx_hbm, ti_hbm, oi_hbm, dist_hbm, out_hbm,
                 sidx_v, tiv, oiv, tidx, oidx, tse, bidx, t_buf, o_buf,
                 tbc, emb, sem_g, sem_w0, sem_w1):
    wid = lax.axis_index("s") * 2 + lax.axis_index("c")
    b = wid // 2
    i_lo = (wid % 2) * _NTH

    pltpu.sync_copy(sidx_hbm, sidx_v)
    pltpu.sync_copy(ti_hbm.at[pl.ds(b * _NT + i_lo, _NTH)], tiv)
    pltpu.sync_copy(oi_hbm.at[pl.ds(b * _NO, _NO)], oiv)

    ti = tiv[...]
    oi0 = oiv[pl.ds(0, 16)]
    oi1 = oiv[pl.ds(16, 16)]

    base = b * _S
    tidx[...] = ti + base
    oidx[pl.ds(0, 16)] = oi0 + base
    oidx[pl.ds(16, 16)] = oi1 + base

    cp_t = pltpu.async_copy(spans2d.at[tidx], t_buf, sem_g)
    cp_o = pltpu.async_copy(spans2d.at[oidx], o_buf, sem_g)

    t_start = plsc.load_gather(sidx_v, [2 * ti])
    t_end = plsc.load_gather(sidx_v, [2 * ti + 1])
    o_start0 = plsc.load_gather(sidx_v, [2 * oi0])
    o_start1 = plsc.load_gather(sidx_v, [2 * oi1])
    o_end0 = plsc.load_gather(sidx_v, [2 * oi0 + 1])
    o_end1 = plsc.load_gather(sidx_v, [2 * oi1 + 1])

    tse[pl.ds(0, 16)] = t_start
    tse[pl.ds(16, 16)] = t_end

    cp_t.wait()
    cp_o.wait()

    out_base = b * (_NT * _NO) + i_lo * _NO
    sems = (sem_w0, sem_w1)

    def pair_step(k, carry):
        for p in (0, 1):
            i = 2 * k + p
            sem_w = sems[p]
            row0 = out_base + i * _NO

            @pl.when(k >= 1)
            def _():
                pltpu.make_async_copy(
                    tbc.at[p], out_hbm.at[pl.ds(row0, _NO), pl.ds(0, _D)],
                    sem_w).wait()
                pltpu.make_async_copy(
                    o_buf, out_hbm.at[pl.ds(row0, _NO), pl.ds(_D, _D)],
                    sem_w).wait()
                pltpu.make_async_copy(
                    emb.at[p], out_hbm.at[pl.ds(row0, _NO), pl.ds(2 * _D, _DD)],
                    sem_w).wait()

            for c in range(_D // 16):
                v = t_buf[i, pl.ds(c * 16, 16)]
                for j in range(_NO):
                    tbc[p, j, pl.ds(c * 16, 16)] = v

            fi = jnp.full((16,), i, jnp.int32)
            a_s = plsc.load_gather(tse, [fi])
            b_s = plsc.load_gather(tse, [fi + 16])
            for h, (o_s, o_e) in enumerate(
                    ((o_start0, o_end0), (o_start1, o_end1))):
                md = jnp.minimum(jnp.abs(b_s - o_s), jnp.abs(a_s - o_e))
                bk = jnp.zeros((16,), jnp.int32)
                for t in _BINS:
                    bk = bk + (md >= t).astype(jnp.int32)
                bidx[p, pl.ds(h * 16, 16)] = bk

            ge = pltpu.async_copy(dist_hbm.at[bidx.at[p]], emb.at[p], sem_g)
            pltpu.async_copy(
                tbc.at[p], out_hbm.at[pl.ds(row0, _NO), pl.ds(0, _D)], sem_w)
            pltpu.async_copy(
                o_buf, out_hbm.at[pl.ds(row0, _NO), pl.ds(_D, _D)], sem_w)
            ge.wait()
            pltpu.async_copy(
                emb.at[p], out_hbm.at[pl.ds(row0, _NO), pl.ds(2 * _D, _DD)],
                sem_w)
        return carry

    lax.fori_loop(0, _NTH // 2, pair_step, 0)

    for p in (0, 1):
        sem_w = sems[p]
        pltpu.make_async_copy(
            tbc.at[p], out_hbm.at[pl.ds(out_base, _NO), pl.ds(0, _D)],
            sem_w).wait()
        pltpu.make_async_copy(
            o_buf, out_hbm.at[pl.ds(out_base, _NO), pl.ds(_D, _D)],
            sem_w).wait()
        pltpu.make_async_copy(
            emb.at[p], out_hbm.at[pl.ds(out_base, _NO), pl.ds(2 * _D, _DD)],
            sem_w).wait()


def kernel(spans, span_indices, target_indices, opinion_indices, dist_table):
    spans2d = spans.reshape(_B * _S, _D)
    ti = target_indices.reshape(-1).astype(jnp.int32)
    oi = opinion_indices.reshape(-1).astype(jnp.int32)
    sidx = span_indices.reshape(-1).astype(jnp.int32)
    out = _pair_rep_sc(spans2d, sidx, ti, oi, dist_table)
    return out.reshape(_B, _NT * _NO, _ROW)


# speedup vs baseline: 3.2374x; 3.2374x over previous
import functools

import jax
import jax.numpy as jnp
from jax import lax
from jax.experimental import pallas as pl
from jax.experimental.pallas import tpu as pltpu
from jax.experimental.pallas import tpu_sc as plsc

_B, _S, _D = 16, 4096, 512
_NT = 32
_NO = 32
_NTH = 16
_DD = 128
_ROW = 2 * _D + _DD
_TASK = _NO * _ROW
_BINS = (1, 2, 3, 4, 5, 8, 16, 32, 64)

_mesh = plsc.VectorSubcoreMesh(core_axis_name="c", subcore_axis_name="s")


@functools.partial(
    pl.kernel,
    mesh=_mesh,
    compiler_params=pltpu.CompilerParams(needs_layout_passes=False),
    out_type=jax.ShapeDtypeStruct((_B * _NT * _NO * _ROW,), jnp.float32),
    scratch_types=[
        pltpu.VMEM((2 * _S,), jnp.int32),
        pltpu.VMEM((_NTH,), jnp.int32),
        pltpu.VMEM((_NO,), jnp.int32),
        pltpu.VMEM((_NTH,), jnp.int32),
        pltpu.VMEM((_NO,), jnp.int32),
        pltpu.VMEM((2 * _NTH,), jnp.int32),
        pltpu.VMEM((10 * _DD,), jnp.float32),
        pltpu.VMEM((_NTH, _D), jnp.float32),
        pltpu.VMEM((_NO, _D), jnp.float32),
        pltpu.VMEM((2 * _TASK,), jnp.float32),
        pltpu.SemaphoreType.DMA,
        pltpu.SemaphoreType.DMA,
        pltpu.SemaphoreType.DMA,
    ],
)
def _pair_rep_sc(spans2d, sidx_hbm, ti_hbm, oi_hbm, dist_hbm, out_hbm,
                 sidx_v, tiv, oiv, tidx, oidx, tse, dist_v, t_buf, o_buf,
                 blk, sem_g, sem_w0, sem_w1):
    wid = lax.axis_index("s") * 2 + lax.axis_index("c")
    b = wid // 2
    i_lo = (wid % 2) * _NTH

    pltpu.sync_copy(sidx_hbm, sidx_v)
    pltpu.sync_copy(dist_hbm, dist_v)
    pltpu.sync_copy(ti_hbm.at[pl.ds(b * _NT + i_lo, _NTH)], tiv)
    pltpu.sync_copy(oi_hbm.at[pl.ds(b * _NO, _NO)], oiv)

    ti = tiv[...]
    oi0 = oiv[pl.ds(0, 16)]
    oi1 = oiv[pl.ds(16, 16)]

    base = b * _S
    tidx[...] = ti + base
    oidx[pl.ds(0, 16)] = oi0 + base
    oidx[pl.ds(16, 16)] = oi1 + base

    cp_t = pltpu.async_copy(spans2d.at[tidx], t_buf, sem_g)
    cp_o = pltpu.async_copy(spans2d.at[oidx], o_buf, sem_g)

    t_start = plsc.load_gather(sidx_v, [2 * ti])
    t_end = plsc.load_gather(sidx_v, [2 * ti + 1])
    o_start0 = plsc.load_gather(sidx_v, [2 * oi0])
    o_start1 = plsc.load_gather(sidx_v, [2 * oi1])
    o_end0 = plsc.load_gather(sidx_v, [2 * oi0 + 1])
    o_end1 = plsc.load_gather(sidx_v, [2 * oi1 + 1])

    tse[pl.ds(0, 16)] = t_start
    tse[pl.ds(16, 16)] = t_end

    cp_t.wait()
    cp_o.wait()

    def opj(j, carry):
        for c in range(_D // 16):
            v = o_buf[j, pl.ds(c * 16, 16)]
            for p in (0, 1):
                blk[pl.ds(p * _TASK + j * _ROW + _D + c * 16, 16)] = v
        return carry

    lax.fori_loop(0, _NO, opj, 0)

    out_base = (b * (_NT * _NO) + i_lo * _NO) * _ROW
    sems = (sem_w0, sem_w1)
    lanes = lax.iota(jnp.int32, 16)

    def pair_step(k, carry):
        for p in (0, 1):
            i = 2 * k + p
            sem_w = sems[p]
            off = out_base + i * _TASK

            @pl.when(k >= 1)
            def _():
                pltpu.make_async_copy(
                    blk.at[pl.ds(p * _TASK, _TASK)],
                    out_hbm.at[pl.ds(off, _TASK)], sem_w).wait()

            for c in range(_D // 16):
                v = t_buf[i, pl.ds(c * 16, 16)]
                for j in range(_NO):
                    blk[pl.ds(p * _TASK + j * _ROW + c * 16, 16)] = v

            fi = jnp.full((16,), i, jnp.int32)
            a_s = plsc.load_gather(tse, [fi])
            b_s = plsc.load_gather(tse, [fi + 16])
            for h, (o_s, o_e) in enumerate(
                    ((o_start0, o_end0), (o_start1, o_end1))):
                md = jnp.minimum(jnp.abs(b_s - o_s), jnp.abs(a_s - o_e))
                bk = jnp.zeros((16,), jnp.int32)
                for t in _BINS:
                    bk = bk + (md >= t).astype(jnp.int32)
                src = bk * _DD
                dst = p * _TASK + (h * 16 + lanes) * _ROW + 2 * _D
                for c in range(_DD):
                    vals = plsc.load_gather(dist_v, [src + c])
                    plsc.store_scatter(blk, [dst + c], vals)

            pltpu.async_copy(blk.at[pl.ds(p * _TASK, _TASK)],
                             out_hbm.at[pl.ds(off, _TASK)], sem_w)
        return carry

    lax.fori_loop(0, _NTH // 2, pair_step, 0)

    for p in (0, 1):
        pltpu.make_async_copy(
            blk.at[pl.ds(p * _TASK, _TASK)],
            out_hbm.at[pl.ds(out_base, _TASK)], sems[p]).wait()


def kernel(spans, span_indices, target_indices, opinion_indices, dist_table):
    spans2d = spans.reshape(_B * _S, _D)
    ti = target_indices.reshape(-1).astype(jnp.int32)
    oi = opinion_indices.reshape(-1).astype(jnp.int32)
    sidx = span_indices.reshape(-1).astype(jnp.int32)
    dist = dist_table.reshape(-1)
    out = _pair_rep_sc(spans2d, sidx, ti, oi, dist)
    return out.reshape(_B, _NT * _NO, _ROW)


# speedup vs baseline: 12.5291x; 3.8701x over previous
import functools

import jax
import jax.numpy as jnp
from jax import lax
from jax.experimental import pallas as pl
from jax.experimental.pallas import tpu as pltpu
from jax.experimental.pallas import tpu_sc as plsc

_B, _S, _D = 16, 4096, 512
_NT = 32
_NO = 32
_NTH = 16
_DD = 128
_ROW = 2 * _D + _DD
_BINS = (1, 2, 3, 4, 5, 8, 16, 32, 64)

_mesh = plsc.VectorSubcoreMesh(core_axis_name="c", subcore_axis_name="s")


@functools.partial(
    pl.kernel,
    mesh=_mesh,
    compiler_params=pltpu.CompilerParams(needs_layout_passes=False),
    out_type=(
        jax.ShapeDtypeStruct((_B * _NT, _D), jnp.float32),
        jax.ShapeDtypeStruct((_B * _NO, _D), jnp.float32),
        jax.ShapeDtypeStruct((_B * _NT * _NO,), jnp.int32),
    ),
    scratch_types=[
        pltpu.VMEM((2 * _S,), jnp.int32),
        pltpu.VMEM((_NTH,), jnp.int32),
        pltpu.VMEM((_NO,), jnp.int32),
        pltpu.VMEM((_NTH,), jnp.int32),
        pltpu.VMEM((_NO,), jnp.int32),
        pltpu.VMEM((2 * _NTH,), jnp.int32),
        pltpu.VMEM((_NTH, _D), jnp.float32),
        pltpu.VMEM((_NO, _D), jnp.float32),
        pltpu.VMEM((_NTH * _NO,), jnp.int32),
        pltpu.SemaphoreType.DMA,
        pltpu.SemaphoreType.DMA,
    ],
)
def _gather_stage_sc(spans2d, sidx_hbm, ti_hbm, oi_hbm,
                     tsp_hbm, osp_hbm, bkt_hbm,
                     sidx_v, tiv, oiv, tidx, oidx, tse, t_buf, o_buf, bkv,
                     sem_g, sem_w):
    wid = lax.axis_index("s") * 2 + lax.axis_index("c")
    b = wid // 2
    half = wid % 2
    i_lo = half * _NTH

    pltpu.sync_copy(sidx_hbm, sidx_v)
    pltpu.sync_copy(ti_hbm.at[pl.ds(b * _NT + i_lo, _NTH)], tiv)
    pltpu.sync_copy(oi_hbm.at[pl.ds(b * _NO, _NO)], oiv)

    ti = tiv[...]
    oi0 = oiv[pl.ds(0, 16)]
    oi1 = oiv[pl.ds(16, 16)]

    base = b * _S
    tidx[...] = ti + base
    oidx[pl.ds(0, 16)] = oi0 + base
    oidx[pl.ds(16, 16)] = oi1 + base

    cp_t = pltpu.async_copy(spans2d.at[tidx], t_buf, sem_g)
    cp_o = pltpu.async_copy(spans2d.at[oidx], o_buf, sem_g)

    t_start = plsc.load_gather(sidx_v, [2 * ti])
    t_end = plsc.load_gather(sidx_v, [2 * ti + 1])
    o_start0 = plsc.load_gather(sidx_v, [2 * oi0])
    o_start1 = plsc.load_gather(sidx_v, [2 * oi1])
    o_end0 = plsc.load_gather(sidx_v, [2 * oi0 + 1])
    o_end1 = plsc.load_gather(sidx_v, [2 * oi1 + 1])

    tse[pl.ds(0, 16)] = t_start
    tse[pl.ds(16, 16)] = t_end

    cp_t.wait()
    cp_o.wait()

    wt = pltpu.async_copy(
        t_buf, tsp_hbm.at[pl.ds(b * _NT + i_lo, _NTH), :], sem_w)
    half_writes_o = half == 0

    @pl.when(half_writes_o)
    def _():
        pltpu.async_copy(o_buf, osp_hbm.at[pl.ds(b * _NO, _NO), :], sem_w)

    def task(i, carry):
        fi = jnp.full((16,), i, jnp.int32)
        a_s = plsc.load_gather(tse, [fi])
        b_s = plsc.load_gather(tse, [fi + 16])
        for h, (o_s, o_e) in enumerate(
                ((o_start0, o_end0), (o_start1, o_end1))):
            md = jnp.minimum(jnp.abs(b_s - o_s), jnp.abs(a_s - o_e))
            bk = jnp.zeros((16,), jnp.int32)
            for t in _BINS:
                bk = bk + (md >= t).astype(jnp.int32)
            bkv[pl.ds(i * _NO + h * 16, 16)] = bk
        return carry

    lax.fori_loop(0, _NTH, task, 0)

    wb = pltpu.async_copy(
        bkv, bkt_hbm.at[pl.ds(b * _NT * _NO + i_lo * _NO, _NTH * _NO)], sem_w)

    wt.wait()
    wb.wait()

    @pl.when(half_writes_o)
    def _():
        pltpu.make_async_copy(
            o_buf, osp_hbm.at[pl.ds(b * _NO, _NO), :], sem_w).wait()


def _assemble_tc(t_ref, o_ref, bk_ref, dist_ref, out_ref):
    t = t_ref[0]
    o = o_ref[0]
    tb = jnp.broadcast_to(t[:, None, :], (_NT, _NO, _D)).reshape(_NT * _NO, _D)
    ob = jnp.broadcast_to(o[None, :, :], (_NT, _NO, _D)).reshape(_NT * _NO, _D)
    bk = bk_ref[0, 0]
    onehot = (bk[:, None] == lax.broadcasted_iota(jnp.int32, (1, 128), 1)
              ).astype(jnp.float32)
    emb = jnp.dot(onehot, dist_ref[...],
                  preferred_element_type=jnp.float32)
    out_ref[0, :, 0:_D] = tb
    out_ref[0, :, _D:2 * _D] = ob
    out_ref[0, :, 2 * _D:_ROW] = emb


def kernel(spans, span_indices, target_indices, opinion_indices, dist_table):
    spans2d = spans.reshape(_B * _S, _D)
    ti = target_indices.reshape(-1).astype(jnp.int32)
    oi = opinion_indices.reshape(-1).astype(jnp.int32)
    sidx = span_indices.reshape(-1).astype(jnp.int32)

    t_sp, o_sp, bkt = _gather_stage_sc(spans2d, sidx, ti, oi)

    dist_pad = jnp.pad(dist_table, ((0, 128 - dist_table.shape[0]), (0, 0)))
    out = pl.pallas_call(
        _assemble_tc,
        grid=(_B,),
        in_specs=[
            pl.BlockSpec((1, _NT, _D), lambda b: (b, 0, 0)),
            pl.BlockSpec((1, _NO, _D), lambda b: (b, 0, 0)),
            pl.BlockSpec((1, 1, _NT * _NO), lambda b: (b, 0, 0)),
            pl.BlockSpec((128, _DD), lambda b: (0, 0)),
        ],
        out_specs=pl.BlockSpec((1, _NT * _NO, _ROW), lambda b: (b, 0, 0)),
        out_shape=jax.ShapeDtypeStruct((_B, _NT * _NO, _ROW), jnp.float32),
    )(
        t_sp.reshape(_B, _NT, _D),
        o_sp.reshape(_B, _NO, _D),
        bkt.reshape(_B, 1, _NT * _NO),
        dist_pad,
    )
    return out
